# Initial kernel scaffold; baseline (speedup 1.0000x reference)
import jax
import jax.numpy as jnp
from jax.experimental import pallas as pl


def kernel(x, edge_index, batch, esm_emb, tabular_feats, gat1_W, gat1_asrc, gat1_adst, gat1_b, gat2_W, gat2_asrc, gat2_adst, gat2_b, lstm_Wih_f, lstm_Whh_f, lstm_bih_f, lstm_bhh_f, lstm_Wih_b, lstm_Whh_b, lstm_bih_b, lstm_bhh_b, tab_W, tab_b, tab_gamma, tab_beta, fus1_W, fus1_b, fus1_gamma, fus1_beta, fus2_W, fus2_b, fus3_W, fus3_b):
    raise NotImplementedError("write your pallas kernel here")


# speedup vs baseline: 36.8476x; 36.8476x over previous
import functools
import math

import jax
import jax.numpy as jnp
from jax import lax
from jax.experimental import pallas as pl
from jax.experimental.pallas import tpu as pltpu
from jax.experimental.pallas import tpu_sc as plsc

N = 10000
NP = 10240
E = 640000
HID = 128
AUG = 144
B = 64
ESM = 480
LH = 64

NW = 32
EW = E // NW
CH = 80
NCH = EW // CH
ROWS_PER_TILE = NP // 16


def _sc_edge_kernel(src_hbm, dst_hbm, s_hbm, d_hbm, haug_hbm, acc_out,
                    s_v, d_v, srcc, dstc, exc, rows_v, zrow, acc_sh):
    cid = lax.axis_index("c")
    sid = lax.axis_index("s")
    wid = sid * 2 + cid

    pltpu.sync_copy(s_hbm, s_v)
    pltpu.sync_copy(d_hbm, d_v)

    def _zbody(i, _):
        r = i // 9
        c = (i % 9) * 16
        zrow[r, pl.ds(c, 16)] = jnp.zeros((16,), jnp.float32)
        return 0
    lax.fori_loop(0, 16 * 9, _zbody, 0)

    def _zcopy(k, _):
        pltpu.sync_copy(zrow, acc_sh.at[pl.ds(sid * ROWS_PER_TILE + k * 16, 16), :])
        return 0
    lax.fori_loop(0, ROWS_PER_TILE // 16, _zcopy, 0)
    plsc.subcore_barrier()

    def _chunk(t, _):
        base = wid * EW + t * CH
        pltpu.sync_copy(src_hbm.at[pl.ds(base, CH)], srcc)
        pltpu.sync_copy(dst_hbm.at[pl.ds(base, CH)], dstc)
        for j in range(CH // 16):
            s16 = srcc[pl.ds(j * 16, 16)]
            d16 = dstc[pl.ds(j * 16, 16)]
            sv = plsc.load_gather(s_v, [s16])
            dv = plsc.load_gather(d_v, [d16])
            e = sv + dv
            e = jnp.where(e >= 0.0, e, 0.2 * e)
            exc[pl.ds(j * 16, 16)] = jnp.exp(e)
        pltpu.sync_copy(haug_hbm.at[srcc], rows_v)

        def _scale(r, _):
            ev = plsc.load_gather(exc, [jnp.full((16,), r, jnp.int32)])
            for j in range(AUG // 16):
                rows_v[r, pl.ds(j * 16, 16)] = rows_v[r, pl.ds(j * 16, 16)] * ev
            return 0
        lax.fori_loop(0, CH, _scale, 0)

        pltpu.sync_copy(rows_v, acc_sh.at[dstc], add=True)
        return 0

    lax.fori_loop(0, NCH, _chunk, 0)
    plsc.subcore_barrier()

    lo = sid * ROWS_PER_TILE
    pltpu.sync_copy(acc_sh.at[pl.ds(lo, ROWS_PER_TILE), :],
                    acc_out.at[cid, pl.ds(lo, ROWS_PER_TILE), :])


def _make_sc_edge():
    mesh = plsc.VectorSubcoreMesh(core_axis_name="c", subcore_axis_name="s")
    return pl.kernel(
        _sc_edge_kernel,
        mesh=mesh,
        compiler_params=pltpu.CompilerParams(needs_layout_passes=False,
                                             use_tc_tiling_on_sc=False),
        out_type=jax.ShapeDtypeStruct((2, NP, AUG), jnp.float32),
        scratch_types=[
            pltpu.VMEM((NP,), jnp.float32),
            pltpu.VMEM((NP,), jnp.float32),
            pltpu.VMEM((CH,), jnp.int32),
            pltpu.VMEM((CH,), jnp.int32),
            pltpu.VMEM((CH,), jnp.float32),
            pltpu.VMEM((CH, AUG), jnp.float32),
            pltpu.VMEM((16, AUG), jnp.float32),
            pltpu.VMEM_SHARED((NP, AUG), jnp.float32),
        ],
    )


def _tc_prep_kernel(x_ref, w_ref, asrc_ref, adst_ref, s_ref, d_ref, aug_ref):
    h = jnp.dot(x_ref[...], w_ref[...], preferred_element_type=jnp.float32)
    s_ref[...] = jnp.sum(h * asrc_ref[...][None, :], axis=1)
    d_ref[...] = jnp.sum(h * adst_ref[...][None, :], axis=1)
    aug_ref[:, pl.ds(0, HID)] = h
    blk = h.shape[0]
    lane = lax.broadcasted_iota(jnp.int32, (blk, AUG - HID), 1)
    aug_ref[:, pl.ds(HID, AUG - HID)] = jnp.where(lane == 0, 1.0, 0.0)


def _tc_prep(x_pad, w, asrc, adst):
    grid = 10
    blk = NP // grid
    return pl.pallas_call(
        _tc_prep_kernel,
        grid=(grid,),
        in_specs=[
            pl.BlockSpec((blk, 128), lambda i: (i, 0)),
            pl.BlockSpec((128, HID), lambda i: (0, 0)),
            pl.BlockSpec((HID,), lambda i: (0,)),
            pl.BlockSpec((HID,), lambda i: (0,)),
        ],
        out_specs=[
            pl.BlockSpec((blk,), lambda i: (i,)),
            pl.BlockSpec((blk,), lambda i: (i,)),
            pl.BlockSpec((blk, AUG), lambda i: (i, 0)),
        ],
        out_shape=[
            jax.ShapeDtypeStruct((NP,), jnp.float32),
            jax.ShapeDtypeStruct((NP,), jnp.float32),
            jax.ShapeDtypeStruct((NP, AUG), jnp.float32),
        ],
    )(x_pad, w, asrc, adst)


def _tc_mid_kernel(acc_ref, b_ref, w_ref, asrc_ref, adst_ref,
                   s_ref, d_ref, aug_ref):
    a = acc_ref[0] + acc_ref[1]
    den = a[:, HID:HID + 1] + 1e-16
    h1 = jnp.maximum(a[:, :HID] / den + b_ref[...][None, :], 0.0)
    h = jnp.dot(h1, w_ref[...], preferred_element_type=jnp.float32)
    s_ref[...] = jnp.sum(h * asrc_ref[...][None, :], axis=1)
    d_ref[...] = jnp.sum(h * adst_ref[...][None, :], axis=1)
    aug_ref[:, pl.ds(0, HID)] = h
    blk = h.shape[0]
    lane = lax.broadcasted_iota(jnp.int32, (blk, AUG - HID), 1)
    aug_ref[:, pl.ds(HID, AUG - HID)] = jnp.where(lane == 0, 1.0, 0.0)


def _tc_mid(acc1, b1, w2, asrc2, adst2):
    grid = 10
    blk = NP // grid
    return pl.pallas_call(
        _tc_mid_kernel,
        grid=(grid,),
        in_specs=[
            pl.BlockSpec((2, blk, AUG), lambda i: (0, i, 0)),
            pl.BlockSpec((HID,), lambda i: (0,)),
            pl.BlockSpec((HID, HID), lambda i: (0, 0)),
            pl.BlockSpec((HID,), lambda i: (0,)),
            pl.BlockSpec((HID,), lambda i: (0,)),
        ],
        out_specs=[
            pl.BlockSpec((blk,), lambda i: (i,)),
            pl.BlockSpec((blk,), lambda i: (i,)),
            pl.BlockSpec((blk, AUG), lambda i: (i, 0)),
        ],
        out_shape=[
            jax.ShapeDtypeStruct((NP,), jnp.float32),
            jax.ShapeDtypeStruct((NP,), jnp.float32),
            jax.ShapeDtypeStruct((NP, AUG), jnp.float32),
        ],
    )(acc1, b1, w2, asrc2, adst2)


_BN_INV = 1.0 / math.sqrt(1.0 + 1e-5)


def _tc_head_kernel(acc_ref, b2_ref, batch_ref, esm_ref,
                    wih_f_ref, bif_ref, bhf_ref,
                    wih_b_ref, bib_ref, bhb_ref,
                    tab_ref, tabw_ref, tabb_ref, tabg_ref, tabbeta_ref,
                    f1w_ref, f1b_ref, f1g_ref, f1beta_ref,
                    f2w_ref, f2b_ref, f3w_ref, f3b_ref,
                    out_ref, sums_ref, cnts_ref):
    i = pl.program_id(0)
    a = acc_ref[0] + acc_ref[1]
    den = a[:, HID:HID + 1] + 1e-16
    h2 = jnp.maximum(a[:, :HID] / den + b2_ref[...][None, :], 0.0)
    blk = h2.shape[0]
    seg = lax.broadcasted_iota(jnp.int32, (blk, B), 1)
    oh = (batch_ref[...][:, None] == seg).astype(jnp.float32)
    part = lax.dot_general(oh, h2, (((0,), (0,)), ((), ())),
                           preferred_element_type=jnp.float32)
    cpart = jnp.sum(oh, axis=0)[:, None] * jnp.ones((1, HID), jnp.float32)

    @pl.when(i == 0)
    def _():
        sums_ref[...] = part
        cnts_ref[...] = cpart

    @pl.when(i > 0)
    def _():
        sums_ref[...] = sums_ref[...] + part
        cnts_ref[...] = cnts_ref[...] + cpart

    @pl.when(i == pl.num_programs(0) - 1)
    def _():
        ge = sums_ref[...] / jnp.maximum(cnts_ref[...], 1.0)

        def lstm(wih, bi, bh):
            g = lax.dot_general(esm_ref[...], wih, (((1,), (1,)), ((), ())),
                                preferred_element_type=jnp.float32)
            g = g + bi[None, :] + bh[None, :]
            gi = g[:, :LH]
            gg = g[:, 2 * LH:3 * LH]
            go = g[:, 3 * LH:]
            c = jax.nn.sigmoid(gi) * jnp.tanh(gg)
            return jax.nn.sigmoid(go) * jnp.tanh(c)

        h_f = lstm(wih_f_ref[...], bif_ref[...], bhf_ref[...])
        h_b = lstm(wih_b_ref[...], bib_ref[...], bhb_ref[...])
        seq = jnp.concatenate([h_f, h_b], axis=1)

        te = jnp.dot(tab_ref[...], tabw_ref[...],
                     preferred_element_type=jnp.float32) + tabb_ref[...][None, :]
        te = te * tabg_ref[...][None, :] * _BN_INV + tabbeta_ref[...][None, :]
        te = jnp.maximum(te, 0.0)

        z = jnp.concatenate([ge, seq, te], axis=1)
        z = jnp.dot(z, f1w_ref[...],
                    preferred_element_type=jnp.float32) + f1b_ref[...][None, :]
        z = z * f1g_ref[...][None, :] * _BN_INV + f1beta_ref[...][None, :]
        z = jnp.maximum(z, 0.0)
        z = jnp.maximum(jnp.dot(z, f2w_ref[...], preferred_element_type=jnp.float32)
                        + f2b_ref[...][None, :], 0.0)
        out_ref[...] = jnp.dot(z, f3w_ref[...],
                               preferred_element_type=jnp.float32) + f3b_ref[...][None, :]


def _tc_head(acc2, b2, batch_pad, esm, wih_f, bif, bhf, wih_b, bib, bhb,
             tab_pad, tabw_pad, tabb, tabg, tabbeta,
             f1w, f1b, f1g, f1beta, f2w, f2b, f3w_pad, f3b_pad):
    grid = 10
    blk = NP // grid
    full = lambda *shape: pl.BlockSpec(shape, lambda i: tuple(0 for _ in shape))
    return pl.pallas_call(
        _tc_head_kernel,
        grid=(grid,),
        in_specs=[
            pl.BlockSpec((2, blk, AUG), lambda i: (0, i, 0)),
            pl.BlockSpec((HID,), lambda i: (0,)),
            pl.BlockSpec((blk,), lambda i: (i,)),
            full(B, ESM),
            full(4 * LH, ESM), full(4 * LH), full(4 * LH),
            full(4 * LH, ESM), full(4 * LH), full(4 * LH),
            full(B, 8), full(8, 64), full(64), full(64), full(64),
            full(320, 256), full(256), full(256), full(256),
            full(256, HID), full(HID), full(HID, HID), full(HID),
        ],
        out_specs=pl.BlockSpec((B, HID), lambda i: (0, 0)),
        out_shape=jax.ShapeDtypeStruct((B, HID), jnp.float32),
        scratch_shapes=[
            pltpu.VMEM((B, HID), jnp.float32),
            pltpu.VMEM((B, HID), jnp.float32),
        ],
    )(acc2, b2, batch_pad, esm, wih_f, bif, bhf, wih_b, bib, bhb,
      tab_pad, tabw_pad, tabb, tabg, tabbeta,
      f1w, f1b, f1g, f1beta, f2w, f2b, f3w_pad, f3b_pad)


def kernel(x, edge_index, batch, esm_emb, tabular_feats,
           gat1_W, gat1_asrc, gat1_adst, gat1_b,
           gat2_W, gat2_asrc, gat2_adst, gat2_b,
           lstm_Wih_f, lstm_Whh_f, lstm_bih_f, lstm_bhh_f,
           lstm_Wih_b, lstm_Whh_b, lstm_bih_b, lstm_bhh_b,
           tab_W, tab_b, tab_gamma, tab_beta,
           fus1_W, fus1_b, fus1_gamma, fus1_beta,
           fus2_W, fus2_b, fus3_W, fus3_b):
    src = edge_index[0]
    dst = edge_index[1]

    x_pad = jnp.zeros((NP, 128), jnp.float32).at[:N, :x.shape[1]].set(x)
    w1_pad = jnp.zeros((128, HID), jnp.float32).at[:x.shape[1], :].set(gat1_W)
    batch_pad = jnp.full((NP,), B, jnp.int32).at[:N].set(batch)
    tab_pad = jnp.zeros((B, 8), jnp.float32).at[:, :7].set(tabular_feats)
    tabw_pad = jnp.zeros((8, 64), jnp.float32).at[:7, :].set(tab_W)
    f3w_pad = jnp.zeros((HID, HID), jnp.float32).at[:, :1].set(fus3_W)
    f3b_pad = jnp.zeros((HID,), jnp.float32).at[:1].set(fus3_b)

    sc_edge = _make_sc_edge()

    s1, d1, aug1 = _tc_prep(x_pad, w1_pad, gat1_asrc, gat1_adst)
    acc1 = sc_edge(src, dst, s1, d1, aug1)
    s2, d2, aug2 = _tc_mid(acc1, gat1_b, gat2_W, gat2_asrc, gat2_adst)
    acc2 = sc_edge(src, dst, s2, d2, aug2)
    out = _tc_head(acc2, gat2_b, batch_pad, esm_emb,
                   lstm_Wih_f, lstm_bih_f, lstm_bhh_f,
                   lstm_Wih_b, lstm_bih_b, lstm_bhh_b,
                   tab_pad, tabw_pad, tab_b, tab_gamma, tab_beta,
                   fus1_W, fus1_b, fus1_gamma, fus1_beta,
                   fus2_W, fus2_b, f3w_pad, f3b_pad)
    return out[:, :1]


# speedup vs baseline: 53.5963x; 1.4545x over previous
import functools
import math

import jax
import jax.numpy as jnp
from jax import lax
from jax.experimental import pallas as pl
from jax.experimental.pallas import tpu as pltpu
from jax.experimental.pallas import tpu_sc as plsc

N = 10000
NP = 10240
E = 640000
HID = 128
AUG = 144
B = 64
ESM = 480
LH = 64

NW = 32
EW = E // NW
CH = 80
NCH = EW // CH
ROWS_PER_TILE = NP // 16


def _sc_edge_kernel(src_hbm, dst_hbm, s_hbm, d_hbm, haug_hbm, acc_out,
                    srcc0, dstc0, sbuf0, dbuf0, exc0, rows0,
                    srcc1, dstc1, sbuf1, dbuf1, exc1, rows1,
                    zrow, acc_sh, sem0, sem1):
    cid = lax.axis_index("c")
    sid = lax.axis_index("s")
    wid = sid * 2 + cid
    ebase = wid * EW

    def _zbody(i, _):
        r = i // 9
        c = (i % 9) * 16
        zrow[r, pl.ds(c, 16)] = jnp.zeros((16,), jnp.float32)
        return 0
    lax.fori_loop(0, 16 * 9, _zbody, 0)

    def _zcopy(k, _):
        pltpu.sync_copy(zrow, acc_sh.at[pl.ds(sid * ROWS_PER_TILE + k * 16, 16), :])
        return 0
    lax.fori_loop(0, ROWS_PER_TILE // 16, _zcopy, 0)
    plsc.subcore_barrier()

    bufs = ((srcc0, dstc0, sbuf0, dbuf0, exc0, rows0, sem0),
            (srcc1, dstc1, sbuf1, dbuf1, exc1, rows1, sem1))

    def _stage(t, b):
        srcc, dstc, sbuf, dbuf, _, rows_v, sem = bufs[b]
        base = ebase + t * CH
        pltpu.sync_copy(src_hbm.at[pl.ds(base, CH)], srcc)
        pltpu.sync_copy(dst_hbm.at[pl.ds(base, CH)], dstc)
        pltpu.async_copy(haug_hbm.at[srcc], rows_v, sem)
        pltpu.async_copy(s_hbm.at[srcc], sbuf, sem)
        pltpu.async_copy(d_hbm.at[dstc], dbuf, sem)

    _stage(0, 0)

    def _pair(t2, _):
        for b in range(2):
            srcc, dstc, sbuf, dbuf, exc, rows_v, sem = bufs[b]
            t = t2 * 2 + b

            @pl.when(t + 1 < NCH)
            def _():
                _stage(t + 1, 1 - b)

            pltpu.make_async_copy(haug_hbm.at[srcc], rows_v, sem).wait()
            pltpu.make_async_copy(s_hbm.at[srcc], sbuf, sem).wait()
            pltpu.make_async_copy(d_hbm.at[dstc], dbuf, sem).wait()

            for j in range(CH // 16):
                e = sbuf[pl.ds(j * 16, 16)] + dbuf[pl.ds(j * 16, 16)]
                e = jnp.where(e >= 0.0, e, 0.2 * e)
                exc[pl.ds(j * 16, 16)] = jnp.exp(e)

            def _scale(r, _):
                ev = plsc.load_gather(exc, [jnp.full((16,), r, jnp.int32)])
                for j in range(AUG // 16):
                    rows_v[r, pl.ds(j * 16, 16)] = rows_v[r, pl.ds(j * 16, 16)] * ev
                return 0
            lax.fori_loop(0, CH, _scale, 0)

            pltpu.sync_copy(rows_v, acc_sh.at[dstc], add=True)
        return 0

    lax.fori_loop(0, NCH // 2, _pair, 0)
    plsc.subcore_barrier()

    lo = sid * ROWS_PER_TILE
    pltpu.sync_copy(acc_sh.at[pl.ds(lo, ROWS_PER_TILE), :],
                    acc_out.at[cid, pl.ds(lo, ROWS_PER_TILE), :])


def _make_sc_edge():
    mesh = plsc.VectorSubcoreMesh(core_axis_name="c", subcore_axis_name="s")
    return pl.kernel(
        _sc_edge_kernel,
        mesh=mesh,
        compiler_params=pltpu.CompilerParams(needs_layout_passes=False,
                                             use_tc_tiling_on_sc=False),
        out_type=jax.ShapeDtypeStruct((2, NP, AUG), jnp.float32),
        scratch_types=(
            [t for _ in range(2)
             for t in (pltpu.VMEM((CH,), jnp.int32),
                       pltpu.VMEM((CH,), jnp.int32),
                       pltpu.VMEM((CH,), jnp.float32),
                       pltpu.VMEM((CH,), jnp.float32),
                       pltpu.VMEM((CH,), jnp.float32),
                       pltpu.VMEM((CH, AUG), jnp.float32))
             ]
            + [pltpu.VMEM((16, AUG), jnp.float32),
               pltpu.VMEM_SHARED((NP, AUG), jnp.float32),
               pltpu.SemaphoreType.DMA,
               pltpu.SemaphoreType.DMA]
        ),
    )


def _tc_prep_kernel(x_ref, w_ref, asrc_ref, adst_ref, s_ref, d_ref, aug_ref):
    h = jnp.dot(x_ref[...], w_ref[...], preferred_element_type=jnp.float32)
    s_ref[...] = jnp.sum(h * asrc_ref[...][None, :], axis=1)
    d_ref[...] = jnp.sum(h * adst_ref[...][None, :], axis=1)
    aug_ref[:, pl.ds(0, HID)] = h
    blk = h.shape[0]
    lane = lax.broadcasted_iota(jnp.int32, (blk, AUG - HID), 1)
    aug_ref[:, pl.ds(HID, AUG - HID)] = jnp.where(lane == 0, 1.0, 0.0)


def _tc_prep(x_pad, w, asrc, adst):
    grid = 10
    blk = NP // grid
    return pl.pallas_call(
        _tc_prep_kernel,
        grid=(grid,),
        in_specs=[
            pl.BlockSpec((blk, 128), lambda i: (i, 0)),
            pl.BlockSpec((128, HID), lambda i: (0, 0)),
            pl.BlockSpec((HID,), lambda i: (0,)),
            pl.BlockSpec((HID,), lambda i: (0,)),
        ],
        out_specs=[
            pl.BlockSpec((blk,), lambda i: (i,)),
            pl.BlockSpec((blk,), lambda i: (i,)),
            pl.BlockSpec((blk, AUG), lambda i: (i, 0)),
        ],
        out_shape=[
            jax.ShapeDtypeStruct((NP,), jnp.float32),
            jax.ShapeDtypeStruct((NP,), jnp.float32),
            jax.ShapeDtypeStruct((NP, AUG), jnp.float32),
        ],
    )(x_pad, w, asrc, adst)


def _tc_mid_kernel(acc_ref, b_ref, w_ref, asrc_ref, adst_ref,
                   s_ref, d_ref, aug_ref):
    a = acc_ref[0] + acc_ref[1]
    den = a[:, HID:HID + 1] + 1e-16
    h1 = jnp.maximum(a[:, :HID] / den + b_ref[...][None, :], 0.0)
    h = jnp.dot(h1, w_ref[...], preferred_element_type=jnp.float32)
    s_ref[...] = jnp.sum(h * asrc_ref[...][None, :], axis=1)
    d_ref[...] = jnp.sum(h * adst_ref[...][None, :], axis=1)
    aug_ref[:, pl.ds(0, HID)] = h
    blk = h.shape[0]
    lane = lax.broadcasted_iota(jnp.int32, (blk, AUG - HID), 1)
    aug_ref[:, pl.ds(HID, AUG - HID)] = jnp.where(lane == 0, 1.0, 0.0)


def _tc_mid(acc1, b1, w2, asrc2, adst2):
    grid = 10
    blk = NP // grid
    return pl.pallas_call(
        _tc_mid_kernel,
        grid=(grid,),
        in_specs=[
            pl.BlockSpec((2, blk, AUG), lambda i: (0, i, 0)),
            pl.BlockSpec((HID,), lambda i: (0,)),
            pl.BlockSpec((HID, HID), lambda i: (0, 0)),
            pl.BlockSpec((HID,), lambda i: (0,)),
            pl.BlockSpec((HID,), lambda i: (0,)),
        ],
        out_specs=[
            pl.BlockSpec((blk,), lambda i: (i,)),
            pl.BlockSpec((blk,), lambda i: (i,)),
            pl.BlockSpec((blk, AUG), lambda i: (i, 0)),
        ],
        out_shape=[
            jax.ShapeDtypeStruct((NP,), jnp.float32),
            jax.ShapeDtypeStruct((NP,), jnp.float32),
            jax.ShapeDtypeStruct((NP, AUG), jnp.float32),
        ],
    )(acc1, b1, w2, asrc2, adst2)


_BN_INV = 1.0 / math.sqrt(1.0 + 1e-5)


def _tc_head_kernel(acc_ref, b2_ref, batch_ref, esm_ref,
                    wih_f_ref, bif_ref, bhf_ref,
                    wih_b_ref, bib_ref, bhb_ref,
                    tab_ref, tabw_ref, tabb_ref, tabg_ref, tabbeta_ref,
                    f1w_ref, f1b_ref, f1g_ref, f1beta_ref,
                    f2w_ref, f2b_ref, f3w_ref, f3b_ref,
                    out_ref, sums_ref, cnts_ref):
    i = pl.program_id(0)
    a = acc_ref[0] + acc_ref[1]
    den = a[:, HID:HID + 1] + 1e-16
    h2 = jnp.maximum(a[:, :HID] / den + b2_ref[...][None, :], 0.0)
    blk = h2.shape[0]
    seg = lax.broadcasted_iota(jnp.int32, (blk, B), 1)
    oh = (batch_ref[...][:, None] == seg).astype(jnp.float32)
    part = lax.dot_general(oh, h2, (((0,), (0,)), ((), ())),
                           preferred_element_type=jnp.float32)
    cpart = jnp.sum(oh, axis=0)[:, None] * jnp.ones((1, HID), jnp.float32)

    @pl.when(i == 0)
    def _():
        sums_ref[...] = part
        cnts_ref[...] = cpart

    @pl.when(i > 0)
    def _():
        sums_ref[...] = sums_ref[...] + part
        cnts_ref[...] = cnts_ref[...] + cpart

    @pl.when(i == pl.num_programs(0) - 1)
    def _():
        ge = sums_ref[...] / jnp.maximum(cnts_ref[...], 1.0)

        def lstm(wih, bi, bh):
            g = lax.dot_general(esm_ref[...], wih, (((1,), (1,)), ((), ())),
                                preferred_element_type=jnp.float32)
            g = g + bi[None, :] + bh[None, :]
            gi = g[:, :LH]
            gg = g[:, 2 * LH:3 * LH]
            go = g[:, 3 * LH:]
            c = jax.nn.sigmoid(gi) * jnp.tanh(gg)
            return jax.nn.sigmoid(go) * jnp.tanh(c)

        h_f = lstm(wih_f_ref[...], bif_ref[...], bhf_ref[...])
        h_b = lstm(wih_b_ref[...], bib_ref[...], bhb_ref[...])
        seq = jnp.concatenate([h_f, h_b], axis=1)

        te = jnp.dot(tab_ref[...], tabw_ref[...],
                     preferred_element_type=jnp.float32) + tabb_ref[...][None, :]
        te = te * tabg_ref[...][None, :] * _BN_INV + tabbeta_ref[...][None, :]
        te = jnp.maximum(te, 0.0)

        z = jnp.concatenate([ge, seq, te], axis=1)
        z = jnp.dot(z, f1w_ref[...],
                    preferred_element_type=jnp.float32) + f1b_ref[...][None, :]
        z = z * f1g_ref[...][None, :] * _BN_INV + f1beta_ref[...][None, :]
        z = jnp.maximum(z, 0.0)
        z = jnp.maximum(jnp.dot(z, f2w_ref[...], preferred_element_type=jnp.float32)
                        + f2b_ref[...][None, :], 0.0)
        out_ref[...] = jnp.dot(z, f3w_ref[...],
                               preferred_element_type=jnp.float32) + f3b_ref[...][None, :]


def _tc_head(acc2, b2, batch_pad, esm, wih_f, bif, bhf, wih_b, bib, bhb,
             tab_pad, tabw_pad, tabb, tabg, tabbeta,
             f1w, f1b, f1g, f1beta, f2w, f2b, f3w_pad, f3b_pad):
    grid = 10
    blk = NP // grid
    full = lambda *shape: pl.BlockSpec(shape, lambda i: tuple(0 for _ in shape))
    return pl.pallas_call(
        _tc_head_kernel,
        grid=(grid,),
        in_specs=[
            pl.BlockSpec((2, blk, AUG), lambda i: (0, i, 0)),
            pl.BlockSpec((HID,), lambda i: (0,)),
            pl.BlockSpec((blk,), lambda i: (i,)),
            full(B, ESM),
            full(4 * LH, ESM), full(4 * LH), full(4 * LH),
            full(4 * LH, ESM), full(4 * LH), full(4 * LH),
            full(B, 8), full(8, 64), full(64), full(64), full(64),
            full(320, 256), full(256), full(256), full(256),
            full(256, HID), full(HID), full(HID, HID), full(HID),
        ],
        out_specs=pl.BlockSpec((B, HID), lambda i: (0, 0)),
        out_shape=jax.ShapeDtypeStruct((B, HID), jnp.float32),
        scratch_shapes=[
            pltpu.VMEM((B, HID), jnp.float32),
            pltpu.VMEM((B, HID), jnp.float32),
        ],
    )(acc2, b2, batch_pad, esm, wih_f, bif, bhf, wih_b, bib, bhb,
      tab_pad, tabw_pad, tabb, tabg, tabbeta,
      f1w, f1b, f1g, f1beta, f2w, f2b, f3w_pad, f3b_pad)


def kernel(x, edge_index, batch, esm_emb, tabular_feats,
           gat1_W, gat1_asrc, gat1_adst, gat1_b,
           gat2_W, gat2_asrc, gat2_adst, gat2_b,
           lstm_Wih_f, lstm_Whh_f, lstm_bih_f, lstm_bhh_f,
           lstm_Wih_b, lstm_Whh_b, lstm_bih_b, lstm_bhh_b,
           tab_W, tab_b, tab_gamma, tab_beta,
           fus1_W, fus1_b, fus1_gamma, fus1_beta,
           fus2_W, fus2_b, fus3_W, fus3_b):
    src = edge_index[0]
    dst = edge_index[1]

    x_pad = jnp.zeros((NP, 128), jnp.float32).at[:N, :x.shape[1]].set(x)
    w1_pad = jnp.zeros((128, HID), jnp.float32).at[:x.shape[1], :].set(gat1_W)
    batch_pad = jnp.full((NP,), B, jnp.int32).at[:N].set(batch)
    tab_pad = jnp.zeros((B, 8), jnp.float32).at[:, :7].set(tabular_feats)
    tabw_pad = jnp.zeros((8, 64), jnp.float32).at[:7, :].set(tab_W)
    f3w_pad = jnp.zeros((HID, HID), jnp.float32).at[:, :1].set(fus3_W)
    f3b_pad = jnp.zeros((HID,), jnp.float32).at[:1].set(fus3_b)

    sc_edge = _make_sc_edge()

    s1, d1, aug1 = _tc_prep(x_pad, w1_pad, gat1_asrc, gat1_adst)
    acc1 = sc_edge(src, dst, s1, d1, aug1)
    s2, d2, aug2 = _tc_mid(acc1, gat1_b, gat2_W, gat2_asrc, gat2_adst)
    acc2 = sc_edge(src, dst, s2, d2, aug2)
    out = _tc_head(acc2, gat2_b, batch_pad, esm_emb,
                   lstm_Wih_f, lstm_bih_f, lstm_bhh_f,
                   lstm_Wih_b, lstm_bih_b, lstm_bhh_b,
                   tab_pad, tabw_pad, tab_b, tab_gamma, tab_beta,
                   fus1_W, fus1_b, fus1_gamma, fus1_beta,
                   fus2_W, fus2_b, f3w_pad, f3b_pad)
    return out[:, :1]
